# baseline (speedup 1.0000x reference)
import functools

import jax
import jax.numpy as jnp
from jax import lax
from jax.experimental import pallas as pl
from jax.experimental.pallas import tpu as pltpu
from jax.experimental.pallas import tpu_sc as plsc

_NUM_COLS = 26
_VOCAB = 1000
_DIM = 128
_BATCH = 16384
_NC = 2
_NS = 16
_NW = _NC * _NS
_CHUNK = _BATCH // _NW
_SUB = 128
_NSUB = _CHUNK // _SUB
_NITEMS = _NUM_COLS * _NSUB
_SLOTS = 7
_SKEW = 5


def _build():
    mesh = plsc.VectorSubcoreMesh(core_axis_name="c", subcore_axis_name="s")

    @functools.partial(
        pl.kernel,
        mesh=mesh,
        out_type=jax.ShapeDtypeStruct((_BATCH, _NUM_COLS * _DIM), jnp.float32),
        scratch_types=[
            pltpu.VMEM((_NUM_COLS, _NSUB, _SUB), jnp.int32),
            pltpu.VMEM((_SLOTS, _SUB, _DIM), jnp.float32),
        ]
        + [pltpu.SemaphoreType.DMA] * (2 * _SLOTS + 1),
    )
    def k(*refs):
        tbls = refs[:_NUM_COLS]
        cols = refs[_NUM_COLS:2 * _NUM_COLS]
        out_hbm, idx_v, rows_v = refs[2 * _NUM_COLS:2 * _NUM_COLS + 3]
        sems = refs[2 * _NUM_COLS + 3:]
        gsem = sems[:_SLOTS]
        osem = sems[_SLOTS:2 * _SLOTS]
        isem = sems[2 * _SLOTS]
        wid = lax.axis_index("s") * _NC + lax.axis_index("c")
        base = wid * _CHUNK

        def idx_copy(t):
            return pltpu.make_async_copy(
                cols[t].at[pl.ds(wid * _NSUB, _NSUB), :], idx_v.at[t], isem
            )

        for t in range(_NUM_COLS):
            idx_copy(t).start()

        def drain_idx():
            pltpu.make_async_copy(
                cols[0].at[pl.ds(0, _NSUB * _NUM_COLS), :], idx_v, isem
            ).wait()

        def pipeline(order):
            seq = [(order[i], rep) for rep in range(_NSUB) for i in range(_NUM_COLS)]

            def gather_copy(k_item):
                t, sub = seq[k_item]
                s = k_item % _SLOTS
                return pltpu.make_async_copy(
                    tbls[t].at[idx_v.at[t, sub]], rows_v.at[s], gsem[s]
                )

            def store_copy(k_item):
                t, sub = seq[k_item]
                s = k_item % _SLOTS
                return pltpu.make_async_copy(
                    rows_v.at[s],
                    out_hbm.at[
                        pl.ds(base + sub * _SUB, _SUB), pl.ds(t * _DIM, _DIM)
                    ],
                    osem[s],
                )

            for k_item in range(_NITEMS + _SKEW):
                if k_item < _NITEMS:
                    if k_item >= _SLOTS:
                        store_copy(k_item - _SLOTS).wait()
                    gather_copy(k_item).start()
                if _SKEW <= k_item < _NITEMS + _SKEW:
                    gather_copy(k_item - _SKEW).wait()
                    store_copy(k_item - _SKEW).start()
            for k_item in range(_NITEMS - _SLOTS, _NITEMS):
                store_copy(k_item).wait()

        drain_idx()
        grp = lax.axis_index("c") * 4 + lax.rem(lax.axis_index("s"), 4)
        for g, phase in enumerate((0, 3, 7, 10, 13, 16, 20, 23)):

            @pl.when(grp == g)
            def _(phase=phase):
                pipeline([(t + phase) % _NUM_COLS for t in range(_NUM_COLS)])

    return k


_GATHER_CACHE = []


def _gather_fn():
    if not _GATHER_CACHE:
        _GATHER_CACHE.append(_build())
    return _GATHER_CACHE[0]


def kernel(col_0, col_1, col_2, col_3, col_4, col_5, col_6, col_7, col_8, col_9, col_10, col_11, col_12, col_13, col_14, col_15, col_16, col_17, col_18, col_19, col_20, col_21, col_22, col_23, col_24, col_25, table_0, table_1, table_2, table_3, table_4, table_5, table_6, table_7, table_8, table_9, table_10, table_11, table_12, table_13, table_14, table_15, table_16, table_17, table_18, table_19, table_20, table_21, table_22, table_23, table_24, table_25):
    cols = [
        col_0, col_1, col_2, col_3, col_4, col_5, col_6, col_7, col_8, col_9,
        col_10, col_11, col_12, col_13, col_14, col_15, col_16, col_17,
        col_18, col_19, col_20, col_21, col_22, col_23, col_24, col_25,
    ]
    cols2d = [c.reshape(_NW * _NSUB, _SUB) for c in cols]
    tables = (
        table_0, table_1, table_2, table_3, table_4, table_5, table_6,
        table_7, table_8, table_9, table_10, table_11, table_12, table_13,
        table_14, table_15, table_16, table_17, table_18, table_19, table_20,
        table_21, table_22, table_23, table_24, table_25,
    )
    return _gather_fn()(*tables, *cols2d)


# speedup vs baseline: 1.0249x; 1.0249x over previous
import functools

import jax
import jax.numpy as jnp
from jax import lax
from jax.experimental import pallas as pl
from jax.experimental.pallas import tpu as pltpu
from jax.experimental.pallas import tpu_sc as plsc

_NUM_COLS = 26
_VOCAB = 1000
_DIM = 128
_BATCH = 16384
_NC = 2
_NS = 16
_NW = _NC * _NS
_CHUNK = _BATCH // _NW
_SUB = 128
_NSUB = _CHUNK // _SUB
_NITEMS = _NUM_COLS * _NSUB
_SLOTS = 7
_SKEW = 5


def _build():
    mesh = plsc.VectorSubcoreMesh(core_axis_name="c", subcore_axis_name="s")

    @functools.partial(
        pl.kernel,
        mesh=mesh,
        out_type=jax.ShapeDtypeStruct((_BATCH, _NUM_COLS * _DIM), jnp.float32),
        scratch_types=[
            pltpu.VMEM((_NUM_COLS, _NSUB, _SUB), jnp.int32),
            pltpu.VMEM((_SLOTS, _SUB, _DIM), jnp.float32),
        ]
        + [pltpu.SemaphoreType.DMA] * (2 * _SLOTS + 1),
    )
    def k(*refs):
        tbls = refs[:_NUM_COLS]
        cols = refs[_NUM_COLS:2 * _NUM_COLS]
        out_hbm, idx_v, rows_v = refs[2 * _NUM_COLS:2 * _NUM_COLS + 3]
        sems = refs[2 * _NUM_COLS + 3:]
        gsem = sems[:_SLOTS]
        osem = sems[_SLOTS:2 * _SLOTS]
        isem = sems[2 * _SLOTS]
        wid = lax.axis_index("s") * _NC + lax.axis_index("c")
        base = wid * _CHUNK

        def idx_copy(t):
            return pltpu.make_async_copy(
                cols[t].at[pl.ds(wid * _NSUB, _NSUB), :], idx_v.at[t], isem
            )

        for t in range(_NUM_COLS):
            idx_copy(t).start()

        def drain_idx():
            pltpu.make_async_copy(
                cols[0].at[pl.ds(0, _NSUB * _NUM_COLS), :], idx_v, isem
            ).wait()

        def pipeline(order):
            seq = [(order[i], rep) for rep in range(_NSUB) for i in range(_NUM_COLS)]

            def gather_copy(k_item):
                t, sub = seq[k_item]
                s = k_item % _SLOTS
                return pltpu.make_async_copy(
                    tbls[t].at[idx_v.at[t, sub]], rows_v.at[s], gsem[s]
                )

            def store_copy(k_item):
                t, sub = seq[k_item]
                s = k_item % _SLOTS
                return pltpu.make_async_copy(
                    rows_v.at[s],
                    out_hbm.at[
                        pl.ds(base + sub * _SUB, _SUB), pl.ds(t * _DIM, _DIM)
                    ],
                    osem[s],
                )

            for k_item in range(_NITEMS + _SKEW):
                if k_item < _NITEMS:
                    if k_item >= _SLOTS:
                        store_copy(k_item - _SLOTS).wait()
                    gather_copy(k_item).start()
                if _SKEW <= k_item < _NITEMS + _SKEW:
                    gather_copy(k_item - _SKEW).wait()
                    store_copy(k_item - _SKEW).start()
            for k_item in range(_NITEMS - _SLOTS, _NITEMS):
                store_copy(k_item).wait()

        drain_idx()
        grp = lax.axis_index("c") * 2 + lax.rem(lax.axis_index("s"), 2)
        for g, phase in enumerate((0, 7, 13, 20)):

            @pl.when(grp == g)
            def _(phase=phase):
                pipeline([(t + phase) % _NUM_COLS for t in range(_NUM_COLS)])

    return k


_GATHER_CACHE = []


def _gather_fn():
    if not _GATHER_CACHE:
        _GATHER_CACHE.append(_build())
    return _GATHER_CACHE[0]


def kernel(col_0, col_1, col_2, col_3, col_4, col_5, col_6, col_7, col_8, col_9, col_10, col_11, col_12, col_13, col_14, col_15, col_16, col_17, col_18, col_19, col_20, col_21, col_22, col_23, col_24, col_25, table_0, table_1, table_2, table_3, table_4, table_5, table_6, table_7, table_8, table_9, table_10, table_11, table_12, table_13, table_14, table_15, table_16, table_17, table_18, table_19, table_20, table_21, table_22, table_23, table_24, table_25):
    cols = [
        col_0, col_1, col_2, col_3, col_4, col_5, col_6, col_7, col_8, col_9,
        col_10, col_11, col_12, col_13, col_14, col_15, col_16, col_17,
        col_18, col_19, col_20, col_21, col_22, col_23, col_24, col_25,
    ]
    cols2d = [c.reshape(_NW * _NSUB, _SUB) for c in cols]
    tables = (
        table_0, table_1, table_2, table_3, table_4, table_5, table_6,
        table_7, table_8, table_9, table_10, table_11, table_12, table_13,
        table_14, table_15, table_16, table_17, table_18, table_19, table_20,
        table_21, table_22, table_23, table_24, table_25,
    )
    return _gather_fn()(*tables, *cols2d)
